# Initial kernel scaffold; baseline (speedup 1.0000x reference)
import jax
import jax.numpy as jnp
from jax.experimental import pallas as pl


def kernel(source, target, source_eqv, target_eqv, featinv, Wq, bq, Wk, bk, Wv, bv, Wm, bm, W1, b1, W2, b2, Wr, br, perms):
    raise NotImplementedError("write your pallas kernel here")


# speedup vs baseline: 3.9984x; 3.9984x over previous
import functools
import math

import jax
import jax.numpy as jnp
from jax.experimental import pallas as pl

F = 32
N = 2048
KTOP = 16
G = 60
NUM_HEADS = 4
HEAD_DIM = 8
EPS = 1e-5
MB = 256
NEG = -1e30


def _row_select_max(sc, lane_n):
    mx = jnp.max(sc, axis=1, keepdims=True)
    is_mx = sc == mx
    idx = jnp.min(jnp.where(is_mx, lane_n, N), axis=1, keepdims=True)
    oh = lane_n == idx
    return idx, oh


def _k1_body(srcT_ref, tgt_ref, tgtT_ref, finvT_ref, tefg_ref,
             WqT_ref, Wk_ref, WvT_ref, WmT_ref, W1T_ref, WrT_ref,
             bq_ref, bk_ref, bv_ref, bm_ref, b1_ref, br_ref, Hm_ref,
             hpre_ref, rterm_ref, nn_ref, tgp_ref):
    srcT = srcT_ref[...]
    tgt = tgt_ref[...]
    lane_n = jax.lax.broadcasted_iota(jnp.int32, (MB, N), 1)

    score = jnp.dot(srcT, tgt)
    idx0, oh0 = _row_select_max(score, lane_n)
    nn_ref[...] = idx0
    sc0 = jnp.where(oh0, NEG, score)
    msk0 = oh0.astype(jnp.float32)

    def body(_, carry):
        sc, msk = carry
        _, oh = _row_select_max(sc, lane_n)
        return jnp.where(oh, NEG, sc), jnp.maximum(msk, oh.astype(jnp.float32))

    _, mskf = jax.lax.fori_loop(0, KTOP - 1, body, (sc0, msk0))
    msk = mskf > 0.5

    qT = jnp.dot(srcT, WqT_ref[...]) + bq_ref[...]
    Kt = jnp.dot(Wk_ref[...], tgt) + bk_ref[...]
    VtT = jnp.dot(tgtT_ref[...], WvT_ref[...]) + bv_ref[...]
    inv_sqrt_d = 1.0 / math.sqrt(HEAD_DIM)
    x = jnp.zeros((MB, F), jnp.float32)
    for h in range(NUM_HEADS):
        hrow = Hm_ref[h:h + 1, :]
        s_h = jnp.dot(qT * hrow, Kt) * inv_sqrt_d
        s_h = jnp.where(msk, s_h, NEG)
        s_h = s_h - jnp.max(s_h, axis=1, keepdims=True)
        e_h = jnp.exp(s_h)
        prob = e_h / jnp.sum(e_h, axis=1, keepdims=True)
        x = x + jnp.dot(prob, VtT) * hrow

    attnT = jnp.dot(x, WmT_ref[...]) + bm_ref[...]
    catT = jnp.concatenate([finvT_ref[...], srcT, attnT], axis=1)
    hpre_ref[...] = jnp.dot(catT, W1T_ref[...]) + b1_ref[...]
    rterm_ref[...] = jnp.dot(catT, WrT_ref[...]) + br_ref[...]

    sub_n = jax.lax.broadcasted_iota(jnp.int32, (N, MB), 0)
    ohT = (sub_n == idx0.reshape(1, MB)).astype(jnp.float32)
    tgp_ref[...] = jnp.dot(tefg_ref[...], ohT)


def _k2_body(seT_ref, tgp_ref, e3t_ref, rt_ref):
    acc = jnp.zeros((G, MB), jnp.float32)

    def body(i, acc):
        a_i = jnp.zeros((G, MB), jnp.float32)
        for f in range(F):
            sfi = seT_ref[f, pl.ds(i, 1), :]
            a_i = a_i + sfi * tgp_ref[pl.ds(f * G, G), :]
        return acc + jnp.dot(e3t_ref[i], a_i)

    rt_ref[...] = jax.lax.fori_loop(0, G, body, acc)


def _k3_body(hpre_ref, rterm_ref, W2T_ref, b2_ref, feat_ref):
    h = hpre_ref[...]
    mean = jnp.mean(h, axis=0, keepdims=True)
    xc = h - mean
    var = jnp.mean(xc * xc, axis=0, keepdims=True)
    hn = xc / jnp.sqrt(var + EPS)
    relu = jnp.maximum(hn, 0.0)
    feat_ref[...] = jnp.dot(relu, W2T_ref[...]) + b2_ref[...] + rterm_ref[...]


def kernel(source, target, source_eqv, target_eqv, featinv,
           Wq, bq, Wk, bk, Wv, bv, Wm, bm, W1, b1, W2, b2, Wr, br, perms):
    f32 = jnp.float32
    src = source.reshape(F, N)
    tgt = target.reshape(F, N)
    srcT = src.T
    tgtT = tgt.T
    finvT = featinv.reshape(F, N).T
    se = source_eqv.reshape(F, N, G)
    te = target_eqv.reshape(F, N, G)
    seT = jnp.transpose(se, (0, 2, 1))
    tefg = jnp.transpose(te, (0, 2, 1)).reshape(F * G, N)

    p2 = perms.reshape(G, G)
    e3t = (p2.T[None, :, :] == jnp.arange(G, dtype=perms.dtype)[:, None, None]
           ).astype(f32)
    hm = (jnp.arange(F, dtype=jnp.int32)[None, :] % NUM_HEADS
          == jnp.arange(NUM_HEADS, dtype=jnp.int32)[:, None]).astype(f32)

    nblk = N // MB
    row = lambda b: b.reshape(1, -1)

    hpre, rterm, nn, tgp = pl.pallas_call(
        _k1_body,
        grid=(nblk,),
        in_specs=[
            pl.BlockSpec((MB, F), lambda i: (i, 0)),
            pl.BlockSpec((F, N), lambda i: (0, 0)),
            pl.BlockSpec((N, F), lambda i: (0, 0)),
            pl.BlockSpec((MB, F), lambda i: (i, 0)),
            pl.BlockSpec((F * G, N), lambda i: (0, 0)),
            pl.BlockSpec((F, F), lambda i: (0, 0)),
            pl.BlockSpec((F, F), lambda i: (0, 0)),
            pl.BlockSpec((F, F), lambda i: (0, 0)),
            pl.BlockSpec((F, F), lambda i: (0, 0)),
            pl.BlockSpec((96, 64), lambda i: (0, 0)),
            pl.BlockSpec((96, F), lambda i: (0, 0)),
            pl.BlockSpec((1, F), lambda i: (0, 0)),
            pl.BlockSpec((F, 1), lambda i: (0, 0)),
            pl.BlockSpec((1, F), lambda i: (0, 0)),
            pl.BlockSpec((1, F), lambda i: (0, 0)),
            pl.BlockSpec((1, 64), lambda i: (0, 0)),
            pl.BlockSpec((1, F), lambda i: (0, 0)),
            pl.BlockSpec((NUM_HEADS, F), lambda i: (0, 0)),
        ],
        out_specs=[
            pl.BlockSpec((MB, 64), lambda i: (i, 0)),
            pl.BlockSpec((MB, F), lambda i: (i, 0)),
            pl.BlockSpec((MB, 1), lambda i: (i, 0)),
            pl.BlockSpec((F * G, MB), lambda i: (0, i)),
        ],
        out_shape=[
            jax.ShapeDtypeStruct((N, 64), f32),
            jax.ShapeDtypeStruct((N, F), f32),
            jax.ShapeDtypeStruct((N, 1), jnp.int32),
            jax.ShapeDtypeStruct((F * G, N), f32),
        ],
    )(srcT, tgt, tgtT, finvT, tefg,
      Wq.T, Wk, Wv.T, Wm.T, W1.T, Wr.T,
      row(bq), bk.reshape(F, 1), row(bv), row(bm), row(b1), row(br), hm)

    rT = pl.pallas_call(
        _k2_body,
        grid=(nblk,),
        in_specs=[
            pl.BlockSpec((F, G, MB), lambda i: (0, 0, i)),
            pl.BlockSpec((F * G, MB), lambda i: (0, i)),
            pl.BlockSpec((G, G, G), lambda i: (0, 0, 0)),
        ],
        out_specs=pl.BlockSpec((G, MB), lambda i: (0, i)),
        out_shape=jax.ShapeDtypeStruct((G, N), f32),
    )(seT, tgp, e3t)

    feat = pl.pallas_call(
        _k3_body,
        in_specs=[
            pl.BlockSpec((N, 64), lambda: (0, 0)),
            pl.BlockSpec((N, F), lambda: (0, 0)),
            pl.BlockSpec((64, F), lambda: (0, 0)),
            pl.BlockSpec((1, F), lambda: (0, 0)),
        ],
        out_specs=pl.BlockSpec((N, F), lambda: (0, 0)),
        out_shape=jax.ShapeDtypeStruct((N, F), f32),
    )(hpre, rterm, W2.T, row(b2))

    feat_out = feat.T.reshape(1, F, N, 1)
    r_out = rT.reshape(1, G, N, 1)
    return (feat_out, r_out)


# speedup vs baseline: 5.1315x; 1.2834x over previous
import functools
import math

import jax
import jax.numpy as jnp
from jax.experimental import pallas as pl

F = 32
N = 2048
KTOP = 16
G = 60
NUM_HEADS = 4
HEAD_DIM = 8
EPS = 1e-5
MB = 256
NEG = -1e30


def _row_select_max(sc, lane_n):
    mx = jnp.max(sc, axis=1, keepdims=True)
    is_mx = sc == mx
    idx = jnp.min(jnp.where(is_mx, lane_n, N), axis=1, keepdims=True)
    oh = lane_n == idx
    return idx, oh


def _k1_body(srcT_ref, tgt_ref, tgtT_ref, finvT_ref, te_ref,
             WqT_ref, Wk_ref, WvT_ref, WmT_ref, W1T_ref, WrT_ref,
             bq_ref, bk_ref, bv_ref, bm_ref, b1_ref, br_ref, Hm_ref,
             hpre_ref, rterm_ref, nn_ref, tgp_ref):
    srcT = srcT_ref[...]
    tgt = tgt_ref[...]
    lane_n = jax.lax.broadcasted_iota(jnp.int32, (MB, N), 1)

    score = jnp.dot(srcT, tgt)
    idx0, oh0 = _row_select_max(score, lane_n)
    nn_ref[...] = idx0
    sc0 = jnp.where(oh0, NEG, score)

    def body(_, sc):
        mx = jnp.max(sc, axis=1, keepdims=True)
        return jnp.where(sc == mx, NEG, sc)

    sc_final = jax.lax.fori_loop(0, KTOP - 1, body, sc0)
    msk = sc_final != score

    qT = jnp.dot(srcT, WqT_ref[...]) + bq_ref[...]
    Kt = jnp.dot(Wk_ref[...], tgt) + bk_ref[...]
    VtT = jnp.dot(tgtT_ref[...], WvT_ref[...]) + bv_ref[...]
    inv_sqrt_d = 1.0 / math.sqrt(HEAD_DIM)
    x = jnp.zeros((MB, F), jnp.float32)
    for h in range(NUM_HEADS):
        hrow = Hm_ref[h:h + 1, :]
        s_h = jnp.dot(qT * hrow, Kt) * inv_sqrt_d
        s_h = jnp.where(msk, s_h, NEG)
        s_h = s_h - jnp.max(s_h, axis=1, keepdims=True)
        e_h = jnp.exp(s_h)
        recip = 1.0 / jnp.sum(e_h, axis=1, keepdims=True)
        x = x + (jnp.dot(e_h, VtT) * recip) * hrow

    attnT = jnp.dot(x, WmT_ref[...]) + bm_ref[...]
    catT = jnp.concatenate([finvT_ref[...], srcT, attnT], axis=1)
    hpre_ref[...] = jnp.dot(catT, W1T_ref[...]) + b1_ref[...]
    rterm_ref[...] = jnp.dot(catT, WrT_ref[...]) + br_ref[...]

    sub_n = jax.lax.broadcasted_iota(jnp.int32, (N, MB), 0)
    ohT = (sub_n == idx0.reshape(1, MB)).astype(jnp.float32)
    dn = (((0,), (0,)), ((), ()))
    for f in range(F):
        tgp_ref[pl.ds(f * G, G), :] = jax.lax.dot_general(
            te_ref[f], ohT, dn)


def _k2_body(seT_ref, tgp_ref, e3t_ref, rt_ref):
    acc = jnp.zeros((G, MB), jnp.float32)

    def body(i, acc):
        parts = [jnp.zeros((G, MB), jnp.float32) for _ in range(4)]
        for f in range(F):
            sfi = seT_ref[f, pl.ds(i, 1), :]
            parts[f % 4] = parts[f % 4] + sfi * tgp_ref[pl.ds(f * G, G), :]
        a_i = (parts[0] + parts[1]) + (parts[2] + parts[3])
        return acc + jnp.dot(e3t_ref[i], a_i)

    rt_ref[...] = jax.lax.fori_loop(0, G, body, acc)


def _k3_body(hpre_ref, rterm_ref, W2T_ref, b2_ref, feat_ref):
    h = hpre_ref[...]
    mean = jnp.mean(h, axis=0, keepdims=True)
    xc = h - mean
    var = jnp.mean(xc * xc, axis=0, keepdims=True)
    hn = xc / jnp.sqrt(var + EPS)
    relu = jnp.maximum(hn, 0.0)
    feat_ref[...] = jnp.dot(relu, W2T_ref[...]) + b2_ref[...] + rterm_ref[...]


def kernel(source, target, source_eqv, target_eqv, featinv,
           Wq, bq, Wk, bk, Wv, bv, Wm, bm, W1, b1, W2, b2, Wr, br, perms):
    f32 = jnp.float32
    src = source.reshape(F, N)
    tgt = target.reshape(F, N)
    srcT = src.T
    tgtT = tgt.T
    finvT = featinv.reshape(F, N).T
    se = source_eqv.reshape(F, N, G)
    te = target_eqv.reshape(F, N, G)
    seT = jnp.transpose(se, (0, 2, 1))

    p2 = perms.reshape(G, G)
    e3t = (p2.T[None, :, :] == jnp.arange(G, dtype=perms.dtype)[:, None, None]
           ).astype(f32)
    hm = (jnp.arange(F, dtype=jnp.int32)[None, :] % NUM_HEADS
          == jnp.arange(NUM_HEADS, dtype=jnp.int32)[:, None]).astype(f32)

    nblk = N // MB
    row = lambda b: b.reshape(1, -1)

    hpre, rterm, nn, tgp = pl.pallas_call(
        _k1_body,
        grid=(nblk,),
        in_specs=[
            pl.BlockSpec((MB, F), lambda i: (i, 0)),
            pl.BlockSpec((F, N), lambda i: (0, 0)),
            pl.BlockSpec((N, F), lambda i: (0, 0)),
            pl.BlockSpec((MB, F), lambda i: (i, 0)),
            pl.BlockSpec((F, N, G), lambda i: (0, 0, 0)),
            pl.BlockSpec((F, F), lambda i: (0, 0)),
            pl.BlockSpec((F, F), lambda i: (0, 0)),
            pl.BlockSpec((F, F), lambda i: (0, 0)),
            pl.BlockSpec((F, F), lambda i: (0, 0)),
            pl.BlockSpec((96, 64), lambda i: (0, 0)),
            pl.BlockSpec((96, F), lambda i: (0, 0)),
            pl.BlockSpec((1, F), lambda i: (0, 0)),
            pl.BlockSpec((F, 1), lambda i: (0, 0)),
            pl.BlockSpec((1, F), lambda i: (0, 0)),
            pl.BlockSpec((1, F), lambda i: (0, 0)),
            pl.BlockSpec((1, 64), lambda i: (0, 0)),
            pl.BlockSpec((1, F), lambda i: (0, 0)),
            pl.BlockSpec((NUM_HEADS, F), lambda i: (0, 0)),
        ],
        out_specs=[
            pl.BlockSpec((MB, 64), lambda i: (i, 0)),
            pl.BlockSpec((MB, F), lambda i: (i, 0)),
            pl.BlockSpec((MB, 1), lambda i: (i, 0)),
            pl.BlockSpec((F * G, MB), lambda i: (0, i)),
        ],
        out_shape=[
            jax.ShapeDtypeStruct((N, 64), f32),
            jax.ShapeDtypeStruct((N, F), f32),
            jax.ShapeDtypeStruct((N, 1), jnp.int32),
            jax.ShapeDtypeStruct((F * G, N), f32),
        ],
    )(srcT, tgt, tgtT, finvT, te,
      Wq.T, Wk, Wv.T, Wm.T, W1.T, Wr.T,
      row(bq), bk.reshape(F, 1), row(bv), row(bm), row(b1), row(br), hm)

    rT = pl.pallas_call(
        _k2_body,
        grid=(nblk,),
        in_specs=[
            pl.BlockSpec((F, G, MB), lambda i: (0, 0, i)),
            pl.BlockSpec((F * G, MB), lambda i: (0, i)),
            pl.BlockSpec((G, G, G), lambda i: (0, 0, 0)),
        ],
        out_specs=pl.BlockSpec((G, MB), lambda i: (0, i)),
        out_shape=jax.ShapeDtypeStruct((G, N), f32),
    )(seT, tgp, e3t)

    feat = pl.pallas_call(
        _k3_body,
        in_specs=[
            pl.BlockSpec((N, 64), lambda: (0, 0)),
            pl.BlockSpec((N, F), lambda: (0, 0)),
            pl.BlockSpec((64, F), lambda: (0, 0)),
            pl.BlockSpec((1, F), lambda: (0, 0)),
        ],
        out_specs=pl.BlockSpec((N, F), lambda: (0, 0)),
        out_shape=jax.ShapeDtypeStruct((N, F), f32),
    )(hpre, rterm, W2.T, row(b2))

    feat_out = feat.T.reshape(1, F, N, 1)
    r_out = rT.reshape(1, G, N, 1)
    return (feat_out, r_out)


# speedup vs baseline: 5.1635x; 1.0062x over previous
import functools
import math

import jax
import jax.numpy as jnp
from jax import lax
from jax.experimental import pallas as pl
from jax.experimental.pallas import tpu as pltpu
from jax.experimental.pallas import tpu_sc as plsc

F = 32
N = 2048
KTOP = 16
G = 60
NUM_HEADS = 4
HEAD_DIM = 8
EPS = 1e-5
MB = 256
NEG = -1e30


def _row_select_max(sc, lane_n):
    mx = jnp.max(sc, axis=1, keepdims=True)
    is_mx = sc == mx
    idx = jnp.min(jnp.where(is_mx, lane_n, N), axis=1, keepdims=True)
    oh = lane_n == idx
    return idx, oh


def _k1_body(srcT_ref, tgt_ref, tgtT_ref, finvT_ref,
             WqT_ref, Wk_ref, WvT_ref, WmT_ref, W1T_ref, WrT_ref,
             bq_ref, bk_ref, bv_ref, bm_ref, b1_ref, br_ref, Hm_ref,
             hpre_ref, rterm_ref, nn_ref):
    srcT = srcT_ref[...]
    tgt = tgt_ref[...]
    lane_n = jax.lax.broadcasted_iota(jnp.int32, (MB, N), 1)

    score = jnp.dot(srcT, tgt)
    idx0, oh0 = _row_select_max(score, lane_n)
    nn_ref[...] = idx0
    sc0 = jnp.where(oh0, NEG, score)

    def body(_, sc):
        mx = jnp.max(sc, axis=1, keepdims=True)
        return jnp.where(sc == mx, NEG, sc)

    sc_final = jax.lax.fori_loop(0, KTOP - 1, body, sc0)
    msk = sc_final != score

    qT = jnp.dot(srcT, WqT_ref[...]) + bq_ref[...]
    Kt = jnp.dot(Wk_ref[...], tgt) + bk_ref[...]
    VtT = jnp.dot(tgtT_ref[...], WvT_ref[...]) + bv_ref[...]
    inv_sqrt_d = 1.0 / math.sqrt(HEAD_DIM)
    x = jnp.zeros((MB, F), jnp.float32)
    for h in range(NUM_HEADS):
        hrow = Hm_ref[h:h + 1, :]
        s_h = jnp.dot(qT * hrow, Kt) * inv_sqrt_d
        s_h = jnp.where(msk, s_h, NEG)
        s_h = s_h - jnp.max(s_h, axis=1, keepdims=True)
        e_h = jnp.exp(s_h)
        recip = 1.0 / jnp.sum(e_h, axis=1, keepdims=True)
        x = x + (jnp.dot(e_h, VtT) * recip) * hrow

    attnT = jnp.dot(x, WmT_ref[...]) + bm_ref[...]
    catT = jnp.concatenate([finvT_ref[...], srcT, attnT], axis=1)
    hpre_ref[...] = jnp.dot(catT, W1T_ref[...]) + b1_ref[...]
    rterm_ref[...] = jnp.dot(catT, WrT_ref[...]) + br_ref[...]


_MPW = N // 32


def _sc_gather_body(tef_hbm, idx_hbm, out_hbm, idx_v, rows_v, sem):
    wid = lax.axis_index("s") * 2 + lax.axis_index("c")
    base = wid * _MPW
    pltpu.sync_copy(idx_hbm.at[pl.ds(base, _MPW)], idx_v)
    pltpu.async_copy(tef_hbm.at[idx_v], rows_v, sem).wait()
    pltpu.sync_copy(rows_v, out_hbm.at[pl.ds(base, _MPW)])


def _k2_body(se_ref, tg_ref, e3t_ref, eye_ref, rt_ref, seT_s, tgT_s):
    dn0 = (((0,), (0,)), ((), ()))
    eye = eye_ref[...]
    for f in range(F):
        seT_s[pl.ds(f * G, G), :] = lax.dot_general(se_ref[f], eye, dn0)
        tgT_s[pl.ds(f * G, G), :] = lax.dot_general(
            tg_ref[:, pl.ds(f * G, G)], eye, dn0)

    def body(i, acc):
        parts = [jnp.zeros((G, MB), jnp.float32) for _ in range(4)]
        for f in range(F):
            sfi = seT_s[pl.ds(f * G + i, 1), :]
            parts[f % 4] = parts[f % 4] + sfi * tgT_s[pl.ds(f * G, G), :]
        a_i = (parts[0] + parts[1]) + (parts[2] + parts[3])
        return acc + jnp.dot(e3t_ref[i], a_i)

    rt_ref[...] = jax.lax.fori_loop(0, G, body, jnp.zeros((G, MB), jnp.float32))


def _k3_body(hpre_ref, rterm_ref, W2T_ref, b2_ref, feat_ref):
    h = hpre_ref[...]
    mean = jnp.mean(h, axis=0, keepdims=True)
    xc = h - mean
    var = jnp.mean(xc * xc, axis=0, keepdims=True)
    hn = xc / jnp.sqrt(var + EPS)
    relu = jnp.maximum(hn, 0.0)
    feat_ref[...] = jnp.dot(relu, W2T_ref[...]) + b2_ref[...] + rterm_ref[...]


def kernel(source, target, source_eqv, target_eqv, featinv,
           Wq, bq, Wk, bk, Wv, bv, Wm, bm, W1, b1, W2, b2, Wr, br, perms):
    f32 = jnp.float32
    src = source.reshape(F, N)
    tgt = target.reshape(F, N)
    srcT = src.T
    tgtT = tgt.T
    finvT = featinv.reshape(F, N).T
    se = source_eqv.reshape(F, N, G)
    te = target_eqv.reshape(F, N, G)

    p2 = perms.reshape(G, G)
    e3t = (p2.T[None, :, :] == jnp.arange(G, dtype=perms.dtype)[:, None, None]
           ).astype(f32)
    hm = (jnp.arange(F, dtype=jnp.int32)[None, :] % NUM_HEADS
          == jnp.arange(NUM_HEADS, dtype=jnp.int32)[:, None]).astype(f32)

    nblk = N // MB
    row = lambda b: b.reshape(1, -1)

    hpre, rterm, nn = pl.pallas_call(
        _k1_body,
        grid=(nblk,),
        in_specs=[
            pl.BlockSpec((MB, F), lambda i: (i, 0)),
            pl.BlockSpec((F, N), lambda i: (0, 0)),
            pl.BlockSpec((N, F), lambda i: (0, 0)),
            pl.BlockSpec((MB, F), lambda i: (i, 0)),
            pl.BlockSpec((F, F), lambda i: (0, 0)),
            pl.BlockSpec((F, F), lambda i: (0, 0)),
            pl.BlockSpec((F, F), lambda i: (0, 0)),
            pl.BlockSpec((F, F), lambda i: (0, 0)),
            pl.BlockSpec((96, 64), lambda i: (0, 0)),
            pl.BlockSpec((96, F), lambda i: (0, 0)),
            pl.BlockSpec((1, F), lambda i: (0, 0)),
            pl.BlockSpec((F, 1), lambda i: (0, 0)),
            pl.BlockSpec((1, F), lambda i: (0, 0)),
            pl.BlockSpec((1, F), lambda i: (0, 0)),
            pl.BlockSpec((1, 64), lambda i: (0, 0)),
            pl.BlockSpec((1, F), lambda i: (0, 0)),
            pl.BlockSpec((NUM_HEADS, F), lambda i: (0, 0)),
        ],
        out_specs=[
            pl.BlockSpec((MB, 64), lambda i: (i, 0)),
            pl.BlockSpec((MB, F), lambda i: (i, 0)),
            pl.BlockSpec((MB, 1), lambda i: (i, 0)),
        ],
        out_shape=[
            jax.ShapeDtypeStruct((N, 64), f32),
            jax.ShapeDtypeStruct((N, F), f32),
            jax.ShapeDtypeStruct((N, 1), jnp.int32),
        ],
    )(srcT, tgt, tgtT, finvT,
      Wq.T, Wk, Wv.T, Wm.T, W1.T, Wr.T,
      row(bq), bk.reshape(F, 1), row(bv), row(bm), row(b1), row(br), hm)

    tef = jnp.transpose(te, (1, 0, 2)).reshape(N, F * G)
    sc_gather = functools.partial(
        pl.kernel,
        out_type=jax.ShapeDtypeStruct((N, F * G), f32),
        mesh=plsc.VectorSubcoreMesh(core_axis_name="c", subcore_axis_name="s"),
        scratch_types=[
            pltpu.VMEM((_MPW,), jnp.int32),
            pltpu.VMEM((_MPW, F * G), f32),
            pltpu.SemaphoreType.DMA,
        ],
    )(_sc_gather_body)
    tg = sc_gather(tef, nn.reshape(-1))

    eye = jnp.eye(MB, dtype=f32)
    rT = pl.pallas_call(
        _k2_body,
        grid=(nblk,),
        in_specs=[
            pl.BlockSpec((F, MB, G), lambda i: (0, i, 0)),
            pl.BlockSpec((MB, F * G), lambda i: (i, 0)),
            pl.BlockSpec((G, G, G), lambda i: (0, 0, 0)),
            pl.BlockSpec((MB, MB), lambda i: (0, 0)),
        ],
        out_specs=pl.BlockSpec((G, MB), lambda i: (0, i)),
        out_shape=jax.ShapeDtypeStruct((G, N), f32),
        scratch_shapes=[
            pltpu.VMEM((F * G, MB), f32),
            pltpu.VMEM((F * G, MB), f32),
        ],
    )(se, tg, e3t, eye)

    feat = pl.pallas_call(
        _k3_body,
        in_specs=[
            pl.BlockSpec((N, 64), lambda: (0, 0)),
            pl.BlockSpec((N, F), lambda: (0, 0)),
            pl.BlockSpec((64, F), lambda: (0, 0)),
            pl.BlockSpec((1, F), lambda: (0, 0)),
        ],
        out_specs=pl.BlockSpec((N, F), lambda: (0, 0)),
        out_shape=jax.ShapeDtypeStruct((N, F), f32),
    )(hpre, rterm, W2.T, row(b2))

    feat_out = feat.T.reshape(1, F, N, 1)
    r_out = rT.reshape(1, G, N, 1)
    return (feat_out, r_out)
